# baseline (speedup 1.0000x reference)
import functools

import jax
import jax.numpy as jnp
import numpy as np
from jax.experimental import pallas as pl
from jax.experimental.pallas import tpu as pltpu

_EOS = 3
_NCMD = 6
_NARGS = 16
_ADIM = 257
_EW3 = float(np.exp(-6.0))


def _loss_body(t_ref, cmd_ref, cl_ref, x_ref, out_cmd_ref, out_args_ref,
               acc_ref):
    i = pl.program_id(0)
    n = pl.num_programs(0)
    x = x_ref[...]
    t = t_ref[...] + 1

    e = jnp.exp(x)
    s = jnp.sum(e, axis=-1)

    c = jax.lax.broadcasted_iota(jnp.int32, x.shape, 3)
    ad = jnp.abs(c - t[..., None])
    w = jnp.where(ad <= 3, jnp.exp(-2.0 * ad.astype(jnp.float32)), 0.0)
    z = jnp.sum(w, axis=-1)
    g = jnp.sum(w * x, axis=-1)

    tf = t.astype(jnp.float32)
    delta = jnp.where(t >= 253, _EW3 - jnp.exp(-2.0 * (256.0 - tf)), 0.0)
    z = z + delta
    g = g + delta * x[..., 256]

    cmdf = cmd_ref[...][0]
    cmdb = cmdf[..., None]
    a = jax.lax.broadcasted_iota(jnp.int32, t.shape, 2)
    mask = (((cmdb == 0) & (a < 2)) |
            ((cmdb == 1) & (a < 4)) |
            ((cmdb == 2) & ((a < 2) | (a == 4))) |
            ((cmdb == 5) & (a >= 5))).astype(jnp.float32)

    la = jnp.sum(mask * (jnp.log(s) - g / z))
    da = jnp.sum(mask)

    cl = cl_ref[...]
    eos = (cmdf == _EOS).astype(jnp.float32)
    sdim = cmdf.shape[1]
    r = jax.lax.broadcasted_iota(jnp.int32, (sdim, sdim), 0)
    cc = jax.lax.broadcasted_iota(jnp.int32, (sdim, sdim), 1)
    tri = (r < cc).astype(jnp.float32)
    excl = jnp.dot(eos, tri, preferred_element_type=jnp.float32)
    pad0 = (excl == 0.0).astype(jnp.float32)
    vis = (jnp.sum(eos, axis=1) < float(sdim)).astype(jnp.float32)
    pad = pad0 * vis[:, None]
    mx = jnp.max(cl, axis=-1)
    lse6 = mx + jnp.log(jnp.sum(jnp.exp(cl - mx[..., None]), axis=-1))
    c6 = jax.lax.broadcasted_iota(jnp.int32, cl.shape, 2)
    picked = jnp.sum(jnp.where(c6 == cmdf[..., None], cl, 0.0), axis=-1)
    nll = lse6 - picked
    lc = jnp.sum(pad * nll)
    dc = jnp.sum(pad)

    @pl.when(i == 0)
    def _():
        acc_ref[0] = la
        acc_ref[1] = da
        acc_ref[2] = lc
        acc_ref[3] = dc

    @pl.when(i != 0)
    def _():
        acc_ref[0] += la
        acc_ref[1] += da
        acc_ref[2] += lc
        acc_ref[3] += dc

    @pl.when(i == n - 1)
    def _():
        out_cmd_ref[0, 0] = acc_ref[2] / acc_ref[3]
        out_args_ref[0, 0] = 2.0 * acc_ref[0] / acc_ref[1]


@jax.jit
def kernel(command_logits, args_logits, command, args):
    bsz, sdim = command.shape
    bb = 4
    grid = bsz // bb

    scalar_spec = pl.BlockSpec((1, 1), lambda i: (0, 0),
                               memory_space=pltpu.SMEM)
    oc, oa = pl.pallas_call(
        _loss_body,
        grid=(grid,),
        in_specs=[
            pl.BlockSpec((bb, sdim, _NARGS), lambda i: (i, 0, 0)),
            pl.BlockSpec((1, bb, sdim), lambda i: (i, 0, 0)),
            pl.BlockSpec((bb, sdim, _NCMD), lambda i: (i, 0, 0)),
            pl.BlockSpec((bb, sdim, _NARGS, _ADIM), lambda i: (i, 0, 0, 0)),
        ],
        out_specs=[scalar_spec] * 2,
        out_shape=[jax.ShapeDtypeStruct((1, 1), jnp.float32)] * 2,
        scratch_shapes=[pltpu.SMEM((4,), jnp.float32)],
        compiler_params=pltpu.CompilerParams(
            dimension_semantics=("arbitrary",)),
    )(args, command.reshape(grid, bb, sdim), command_logits, args_logits)

    return (oc[0, 0], oa[0, 0])
